# Initial kernel scaffold; baseline (speedup 1.0000x reference)
import jax
import jax.numpy as jnp
from jax.experimental import pallas as pl


def kernel(x, edge_index, Ww0, bw0, Ww1, bw1, Ww2, bw2, Ws0, bs0, Ws1, bs1, Ws2, bs2, Wg0, bg0, Wg1, bg1):
    raise NotImplementedError("write your pallas kernel here")


# speedup vs baseline: 2.1839x; 2.1839x over previous
import functools

import jax
import jax.numpy as jnp
from jax import lax
from jax.experimental import pallas as pl
from jax.experimental.pallas import tpu as pltpu
from jax.experimental.pallas import tpu_sc as plsc

N = 10000
E = 160000
IN_DIM = 256
HID = 512
OUT = 128

NC = 2
NS = 16
NW = NC * NS
NPAD = 10240
ROWS_PER_TILE = NPAD // NS
EB = 64
EPT = 5120
NEB = EPT // EB
TRASH = NPAD - 1

BM = 1000


def _sc_aggregate(nc, with_deg):
  out_type = [jax.ShapeDtypeStruct((NC * NPAD, 128), jnp.float32)
              for _ in range(nc + (1 if with_deg else 0))]

  scratch = dict(
      src_v=pltpu.VMEM((NEB, EB), jnp.int32),
      dst_v=pltpu.VMEM((NEB, EB), jnp.int32),
      rows_v=pltpu.VMEM((EB, 128), jnp.float32),
      acc=pltpu.VMEM_SHARED((NPAD, 128), jnp.float32),
      sem=pltpu.SemaphoreType.DMA,
  )

  mesh = plsc.VectorSubcoreMesh(core_axis_name="c", subcore_axis_name="s")

  @functools.partial(pl.kernel, out_type=out_type, mesh=mesh,
                     scratch_types=scratch)
  def k(*refs, src_v, dst_v, rows_v, acc, sem):
    tabs = refs[:nc]
    srci, dsti, z640, o128 = refs[nc:nc + 4]
    outs = refs[nc + 4:]
    c = lax.axis_index("c")
    s = lax.axis_index("s")
    w = c * NS + s
    stripe = s * ROWS_PER_TILE
    flat = c * NPAD + stripe

    pltpu.sync_copy(srci.at[w], src_v)
    pltpu.sync_copy(dsti.at[w], dst_v)

    def run_pass(scatter_body, out_ref):
      pltpu.sync_copy(z640, acc.at[pl.ds(stripe, ROWS_PER_TILE)])
      plsc.subcore_barrier()

      @pl.loop(0, NEB)
      def _(j):
        scatter_body(j)

      plsc.subcore_barrier()
      pltpu.sync_copy(acc.at[pl.ds(stripe, ROWS_PER_TILE)],
                      out_ref.at[pl.ds(flat, ROWS_PER_TILE)])
      plsc.subcore_barrier()

    if with_deg:
      pltpu.sync_copy(o128, rows_v)

      def deg_body(j):
        pltpu.sync_copy(rows_v, acc.at[dst_v.at[j]], add=True)

      run_pass(deg_body, outs[nc])

    for ci in range(nc):
      def agg_body(j, tab=tabs[ci]):
        pltpu.async_copy(tab.at[src_v.at[j]], rows_v, sem).wait()
        pltpu.sync_copy(rows_v, acc.at[dst_v.at[j]], add=True)

      run_pass(agg_body, outs[ci])

  return k


_sc_agg1 = _sc_aggregate(2, True)
_sc_agg2 = _sc_aggregate(4, False)
_sc_agg3 = _sc_aggregate(1, False)


def _row_blocks(width):
  return pl.BlockSpec((BM, width), lambda i: (i, 0))


def _full(shape):
  nd = len(shape)
  return pl.BlockSpec(shape, lambda i: (0,) * nd)


def _partial_spec():
  return pl.BlockSpec((NC, BM, 128), lambda i: (0, i, 0))


def _deg_spec():
  return _partial_spec()


def _deg_from(degp):
  d = degp[0, :, 0:1] + degp[1, :, 0:1]
  return jnp.maximum(d, 1.0)


def _weak_body(x, w0, b0, w1, b1, w2, b2, out):
  h = jax.nn.relu(jnp.dot(x[...], w0[...]) + b0[...])
  h = jax.nn.relu(jnp.dot(h, w1[...]) + b1[...])
  out[...] = jnp.dot(h, w2[...]) + b2[...]


def _layer0_body(x, p0, p1, degp, w, b, out):
  deg = _deg_from(degp[...])
  agg = jnp.concatenate([p0[0] + p0[1], p1[0] + p1[1]], axis=1)
  y = x[...] + agg / deg
  g = jax.nn.relu(jnp.dot(y, w[...]) + b[...])
  out[...] = g.reshape(BM, 4, 128).transpose(1, 0, 2)


def _layer1_body(g0t, q0, q1, q2, q3, degp, w1, b1, w2, out):
  deg = _deg_from(degp[...])
  g0 = g0t[...].transpose(1, 0, 2).reshape(BM, HID)
  agg = jnp.concatenate(
      [q[0] + q[1] for q in (q0, q1, q2, q3)], axis=1)
  y = g0 + agg / deg
  g1 = jax.nn.relu(jnp.dot(y, w1[...]) + b1[...])
  out[...] = jnp.dot(g1, w2[...])


def _final_body(z, r, degp, bs2, weak, wg0, bg0, wg1, bg1, out):
  deg = _deg_from(degp[...])
  strong = z[...] + (r[0] + r[1]) / deg + bs2[...]
  wk = weak[...]
  cg = jax.nn.sigmoid(
      jnp.dot(jax.nn.relu(jnp.dot(wk, wg0[...]) + bg0[...]), wg1[...])
      + bg1[...])
  out[...] = cg * wk + (1.0 - cg) * strong


def kernel(x, edge_index, Ww0, bw0, Ww1, bw1, Ww2, bw2,
           Ws0, bs0, Ws1, bs1, Ws2, bs2, Wg0, bg0, Wg1, bg1):
  f32 = jnp.float32
  src = edge_index[0]
  dst = edge_index[1]
  pad = NW * EPT - E
  src_t = jnp.concatenate(
      [src, jnp.zeros((pad,), jnp.int32)]).reshape(NW, NEB, EB)
  dst_t = jnp.concatenate(
      [dst, jnp.full((pad,), TRASH, jnp.int32)]).reshape(NW, NEB, EB)

  z640 = jnp.zeros((ROWS_PER_TILE, 128), f32)
  o128 = jnp.ones((EB, 128), f32)

  xt = x.reshape(N, 2, 128).transpose(1, 0, 2)

  p0, p1, degp = _sc_agg1(xt[0], xt[1], src_t, dst_t, z640, o128)
  p0, p1, degp = (a.reshape(NC, NPAD, 128) for a in (p0, p1, degp))

  bw0_, bw1_, bw2_ = bw0[None], bw1[None], bw2[None]
  bs0_, bs1_, bs2_ = bs0[None], bs1[None], bs2[None]
  bg0_, bg1_ = bg0[None], bg1[None]

  grid = (N // BM,)

  weak = pl.pallas_call(
      _weak_body,
      grid=grid,
      in_specs=[_row_blocks(IN_DIM), _full((IN_DIM, HID)), _full((1, HID)),
                _full((HID, HID)), _full((1, HID)),
                _full((HID, OUT)), _full((1, OUT))],
      out_specs=_row_blocks(OUT),
      out_shape=jax.ShapeDtypeStruct((N, OUT), f32),
  )(x, Ww0, bw0_, Ww1, bw1_, Ww2, bw2_)

  g0t = pl.pallas_call(
      _layer0_body,
      grid=grid,
      in_specs=[_row_blocks(IN_DIM), _partial_spec(), _partial_spec(),
                _deg_spec(), _full((IN_DIM, HID)), _full((1, HID))],
      out_specs=pl.BlockSpec((4, BM, 128), lambda i: (0, i, 0)),
      out_shape=jax.ShapeDtypeStruct((4, N, 128), f32),
  )(x, p0, p1, degp, Ws0, bs0_)

  q0, q1, q2, q3 = (a.reshape(NC, NPAD, 128) for a in _sc_agg2(
      g0t[0], g0t[1], g0t[2], g0t[3], src_t, dst_t, z640, o128))

  z = pl.pallas_call(
      _layer1_body,
      grid=grid,
      in_specs=[pl.BlockSpec((4, BM, 128), lambda i: (0, i, 0)),
                _partial_spec(), _partial_spec(), _partial_spec(),
                _partial_spec(), _deg_spec(),
                _full((HID, HID)), _full((1, HID)), _full((HID, OUT))],
      out_specs=_row_blocks(OUT),
      out_shape=jax.ShapeDtypeStruct((N, OUT), f32),
  )(g0t, q0, q1, q2, q3, degp, Ws1, bs1_, Ws2)

  (r,) = _sc_agg3(z, src_t, dst_t, z640, o128)
  r = r.reshape(NC, NPAD, 128)

  out = pl.pallas_call(
      _final_body,
      grid=grid,
      in_specs=[_row_blocks(OUT), _partial_spec(), _deg_spec(),
                _full((1, OUT)), _row_blocks(OUT),
                _full((OUT, 64)), _full((1, 64)),
                _full((64, 1)), _full((1, 1))],
      out_specs=_row_blocks(OUT),
      out_shape=jax.ShapeDtypeStruct((N, OUT), f32),
  )(z, r, degp, bs2_, weak, Wg0, bg0_, Wg1, bg1_)

  return out
